# Initial kernel scaffold; baseline (speedup 1.0000x reference)
import jax
import jax.numpy as jnp
from jax.experimental import pallas as pl


def kernel(hidden_states, residual, token_mask, prob, counts, state):
    raise NotImplementedError("write your pallas kernel here")


# speedup vs baseline: 7.1135x; 7.1135x over previous
import functools

import jax
import jax.numpy as jnp
from jax import lax
from jax.experimental import pallas as pl
from jax.experimental.pallas import tpu as pltpu


def _detok_kernel(mask_ref, prob_ref, counts_ref, hs_ref, res_ref, state_ref,
                  out_ref, ns_ref, *, L):
    cnt = counts_ref[0, 0, 0]
    z0 = state_ref[0, 0:1, :]

    def body(l, carry):
        c, z, cap = carry
        m = mask_ref[0, 0, l]
        p = prob_ref[0, 0, l]
        c1 = c + m
        decay = jnp.clip(1.0 - p, 0.0, 1.0)
        peff = (1.0 - decay) * m.astype(jnp.float32)
        g = hs_ref[0, pl.ds(jnp.maximum(c1 - 1, 0), 1), :]
        z1 = z + peff * (g - z)
        validf = jnp.where((c1 >= 1) & (c1 <= cnt), 1.0, 0.0)
        out_ref[0, pl.ds(l, 1), :] = res_ref[0, pl.ds(l, 1), :] + validf * z1
        cap1 = jnp.where((m == 1) & (c1 == cnt), z1, cap)
        return (c1, z1, cap1)

    c_fin, z_fin, cap_fin = lax.fori_loop(0, L, body, (jnp.int32(0), z0, z0))
    ns_ref[0, 0:1, :] = jnp.where(cnt > c_fin, z_fin, cap_fin)


@jax.jit
def kernel(hidden_states, residual, token_mask, prob, counts, state):
    B, L, D = residual.shape
    M = hidden_states.shape[1]
    mask_i32 = token_mask.astype(jnp.int32).reshape(B, 1, L)
    prob_3d = prob.reshape(B, 1, L)
    counts_3d = counts.astype(jnp.int32).reshape(B, 1, 1)
    state_3d = state.reshape(B, 1, D)

    out, new_state = pl.pallas_call(
        functools.partial(_detok_kernel, L=L),
        grid=(B,),
        in_specs=[
            pl.BlockSpec((1, 1, L), lambda b: (b, 0, 0), memory_space=pltpu.SMEM),
            pl.BlockSpec((1, 1, L), lambda b: (b, 0, 0), memory_space=pltpu.SMEM),
            pl.BlockSpec((1, 1, 1), lambda b: (b, 0, 0), memory_space=pltpu.SMEM),
            pl.BlockSpec((1, M, D), lambda b: (b, 0, 0)),
            pl.BlockSpec((1, L, D), lambda b: (b, 0, 0)),
            pl.BlockSpec((1, 1, D), lambda b: (b, 0, 0)),
        ],
        out_specs=[
            pl.BlockSpec((1, L, D), lambda b: (b, 0, 0)),
            pl.BlockSpec((1, 1, D), lambda b: (b, 0, 0)),
        ],
        out_shape=[
            jax.ShapeDtypeStruct((B, L, D), jnp.float32),
            jax.ShapeDtypeStruct((B, 1, D), jnp.float32),
        ],
        compiler_params=pltpu.CompilerParams(
            dimension_semantics=("arbitrary",),
            vmem_limit_bytes=64 * 1024 * 1024,
        ),
    )(mask_i32, prob_3d, counts_3d, hidden_states, residual, state_3d)
    return (out, new_state.reshape(B, D))
